# baseline (speedup 1.0000x reference)
import functools

import jax
import jax.numpy as jnp
from jax import lax
from jax.experimental import pallas as pl
from jax.experimental.pallas import tpu as pltpu
from jax.experimental.pallas import tpu_sc as plsc

_POWER = 2
_BATCH, _MAX_LEN = 4096, 50
_VOCAB = 1000000
_EMBED_DIM, _MASK_DIM = 64, 16

_NW = 32
_BPW = _BATCH // _NW
_NUNIT = _MAX_LEN
_PACK_N = 32768
_PACK_B = _PACK_N.bit_length() - 1
_PACK_GRID = (_VOCAB + _PACK_N - 1) // _PACK_N
_PACK_ROWS = _PACK_GRID * (_PACK_N // 2)


def _pack_body(x_ref, o_ref):
    x = x_ref[...]
    xt = jnp.transpose(x, (1, 0))
    o_ref[...] = jnp.concatenate([xt[: _PACK_N // 2], xt[_PACK_N // 2:]],
                                 axis=1)


def _tc_pack(word_table):
    wtT = word_table.T
    return pl.pallas_call(
        _pack_body,
        grid=(_PACK_GRID,),
        in_specs=[pl.BlockSpec((64, _PACK_N), lambda i: (0, i))],
        out_specs=pl.BlockSpec((_PACK_N // 2, 128), lambda i: (i, 0)),
        out_shape=jax.ShapeDtypeStruct((_PACK_ROWS, 128), jnp.float32),
    )(wtT)


def _sc_gather(wt_packed, sent_t):
    mesh = plsc.VectorSubcoreMesh(core_axis_name="c", subcore_axis_name="s")

    @functools.partial(
        pl.kernel,
        mesh=mesh,
        out_type=jax.ShapeDtypeStruct((_MAX_LEN, _EMBED_DIM, _BATCH),
                                      jnp.float32),
        scratch_types=[
            pltpu.VMEM((_NUNIT, _BPW), jnp.int32),
            pltpu.VMEM((_NUNIT, _BPW), jnp.int32),
            pltpu.VMEM((_BPW, 128), jnp.float32),
            pltpu.VMEM((_BPW, 128), jnp.float32),
            pltpu.VMEM((_BPW, 128), jnp.float32),
            pltpu.VMEM((_BPW, 128), jnp.float32),
            pltpu.VMEM((_EMBED_DIM, _BPW), jnp.float32),
            pltpu.VMEM((_EMBED_DIM, _BPW), jnp.float32),
            pltpu.VMEM((_EMBED_DIM, _BPW), jnp.float32),
            pltpu.VMEM((_EMBED_DIM, _BPW), jnp.float32),
        ] + [pltpu.SemaphoreType.DMA] * 8,
        compiler_params=pltpu.CompilerParams(use_tc_tiling_on_sc=True,
                                             needs_layout_passes=False),
    )
    def body(wt_hbm, sidx_hbm, out_hbm,
             idx_v, kidx_v, gbuf0, gbuf1, gbuf2, gbuf3,
             tbuf0, tbuf1, tbuf2, tbuf3,
             g0, g1, g2, g3, w0, w1, w2, w3):
        wid = lax.axis_index("s") * 2 + lax.axis_index("c")
        b0 = wid * _BPW
        pltpu.sync_copy(sidx_hbm.at[:, pl.ds(b0, _BPW)], idx_v)

        def shift_row(j, carry):
            for c in range(_BPW // 16):
                v = idx_v[j, pl.ds(c * 16, 16)]
                kidx_v[j, pl.ds(c * 16, 16)] = (
                    ((v >> _PACK_B) << (_PACK_B - 1))
                    | (v & (_PACK_N // 2 - 1)))
            return carry
        lax.fori_loop(0, _NUNIT, shift_row, 0)

        gbuf = (gbuf0, gbuf1, gbuf2, gbuf3)
        tbuf = (tbuf0, tbuf1, tbuf2, tbuf3)
        gsem = (g0, g1, g2, g3)
        wsem = (w0, w1, w2, w3)

        def start_gather(j, b):
            return pltpu.async_copy(wt_hbm.at[kidx_v.at[j]], gbuf[b], gsem[b])

        def start_writeback(j, b):
            return pltpu.async_copy(
                tbuf[b], out_hbm.at[j, :, pl.ds(b0, _BPW)], wsem[b])

        def wait_gather(j, b):
            pltpu.make_async_copy(wt_hbm.at[kidx_v.at[j]], gbuf[b],
                                  gsem[b]).wait()

        def wait_writeback(j, b):
            pltpu.make_async_copy(
                tbuf[b], out_hbm.at[j, :, pl.ds(b0, _BPW)], wsem[b]).wait()

        def transpose_unit(j, b):
            iota = lax.iota(jnp.int32, 16)
            for c in range(_BPW // 16):
                rows = iota + (c * 16)
                h6 = ((idx_v[j, pl.ds(c * 16, 16)] >> (_PACK_B - 1)) & 1) << 6

                @plsc.parallel_loop(0, _EMBED_DIM, step=1, unroll=16)
                def dloop(d):
                    v = plsc.load_gather(gbuf[b], [rows, h6 + d])
                    tbuf[b][d, pl.ds(c * 16, 16)] = v

        for b in range(4):
            start_gather(b, b)

        def unit_quad(s, carry):
            for b in (0, 1, 2, 3):
                j = 4 * s + b

                @pl.when((j >= 4) & (j < _NUNIT + 4))
                def _():
                    wait_writeback(j - 4, b)

                @pl.when(j < _NUNIT)
                def _():
                    wait_gather(j, b)
                    transpose_unit(j, b)

                @pl.when(j + 4 < _NUNIT)
                def _():
                    start_gather(j + 4, b)

                @pl.when(j < _NUNIT)
                def _():
                    start_writeback(j, b)
            return carry

        lax.fori_loop(0, _NUNIT // 4 + 2, unit_quad, 0)

    return body(wt_packed, sent_t)


def _tc_body(mask_t_ref, mtab_t_ref, m3_ref, pw_ref):
    m = mask_t_ref[...]
    q = lax.broadcasted_iota(jnp.int32, m.shape, 0)
    left = jnp.min(jnp.where(m == 1, q, jnp.int32(1 << 30)), axis=0,
                   keepdims=True)
    right = left + jnp.sum(m, axis=0, keepdims=True)
    d = jnp.where(q < left, left - q, jnp.where(q > right, q - right, 0))
    num = (100 - d) ** _POWER
    pw_ref[...] = num.astype(jnp.float32) / jnp.float32(100 ** _POWER)

    mt = mtab_t_ref[...]
    mt0 = mt[:, 0].reshape(1, _MASK_DIM, 1)
    mt1 = mt[:, 1].reshape(1, _MASK_DIM, 1)
    sel = (m == 1)[:, None, :]
    m3_ref[...] = jnp.where(sel, mt1, mt0)


def _tc_mask_pw(mask_t, mtab_t):
    grid = 8
    bb = _BATCH // grid
    return pl.pallas_call(
        _tc_body,
        grid=(grid,),
        in_specs=[
            pl.BlockSpec((_MAX_LEN, bb), lambda i: (0, i)),
            pl.BlockSpec((_MASK_DIM, 2), lambda i: (0, 0)),
        ],
        out_specs=[
            pl.BlockSpec((_MAX_LEN, _MASK_DIM, bb), lambda i: (0, 0, i)),
            pl.BlockSpec((_MAX_LEN, bb), lambda i: (0, i)),
        ],
        out_shape=[
            jax.ShapeDtypeStruct((_MAX_LEN, _MASK_DIM, _BATCH), jnp.float32),
            jax.ShapeDtypeStruct((_MAX_LEN, _BATCH), jnp.float32),
        ],
    )(mask_t, mtab_t)


def kernel(sent, mask, word_table, mask_table):
    wt_packed = _tc_pack(word_table)
    sent_t = sent.T
    mask_t = mask.T
    mtab_t = mask_table.T
    w3 = _sc_gather(wt_packed, sent_t)
    m3, pw_t = _tc_mask_pw(mask_t, mtab_t)
    sent_vec = jnp.transpose(w3, (2, 0, 1))
    mask_vec = jnp.transpose(m3, (2, 0, 1))
    position_weight = pw_t.T
    return (sent_vec, mask_vec, position_weight)


# speedup vs baseline: 1.0021x; 1.0021x over previous
import functools

import jax
import jax.numpy as jnp
from jax import lax
from jax.experimental import pallas as pl
from jax.experimental.pallas import tpu as pltpu
from jax.experimental.pallas import tpu_sc as plsc

_POWER = 2
_BATCH, _MAX_LEN = 4096, 50
_VOCAB = 1000000
_EMBED_DIM, _MASK_DIM = 64, 16

_NW = 32
_BPW = _BATCH // _NW
_NUNIT = _MAX_LEN
_PACK_N = 32768
_PACK_B = _PACK_N.bit_length() - 1
_PACK_GRID = (_VOCAB + _PACK_N - 1) // _PACK_N
_PACK_ROWS = _PACK_GRID * (_PACK_N // 2)


def _pack_body(x_ref, o_ref):
    x = x_ref[...]
    xt = jnp.transpose(x, (1, 0))
    o_ref[...] = jnp.concatenate([xt[: _PACK_N // 2], xt[_PACK_N // 2:]],
                                 axis=1)


def _tc_pack(word_table):
    wtT = word_table.T
    return pl.pallas_call(
        _pack_body,
        grid=(_PACK_GRID,),
        in_specs=[pl.BlockSpec((64, _PACK_N), lambda i: (0, i))],
        out_specs=pl.BlockSpec((_PACK_N // 2, 128), lambda i: (i, 0)),
        out_shape=jax.ShapeDtypeStruct((_PACK_ROWS, 128), jnp.float32),
    )(wtT)


def _sc_gather(wt_packed, sent_t):
    mesh = plsc.VectorSubcoreMesh(core_axis_name="c", subcore_axis_name="s")

    @functools.partial(
        pl.kernel,
        mesh=mesh,
        out_type=jax.ShapeDtypeStruct((_MAX_LEN, _EMBED_DIM, _BATCH),
                                      jnp.float32),
        scratch_types=[
            pltpu.VMEM((_NUNIT, _BPW), jnp.int32),
            pltpu.VMEM((_NUNIT, _BPW), jnp.int32),
            pltpu.VMEM((_BPW, 128), jnp.float32),
            pltpu.VMEM((_BPW, 128), jnp.float32),
            pltpu.VMEM((_BPW, 128), jnp.float32),
            pltpu.VMEM((_BPW, 128), jnp.float32),
            pltpu.VMEM((_EMBED_DIM, _BPW), jnp.float32),
            pltpu.VMEM((_EMBED_DIM, _BPW), jnp.float32),
            pltpu.VMEM((_EMBED_DIM, _BPW), jnp.float32),
            pltpu.VMEM((_EMBED_DIM, _BPW), jnp.float32),
        ] + [pltpu.SemaphoreType.DMA] * 8,
        compiler_params=pltpu.CompilerParams(use_tc_tiling_on_sc=True,
                                             needs_layout_passes=False),
    )
    def body(wt_hbm, sidx_hbm, out_hbm,
             idx_v, kidx_v, gbuf0, gbuf1, gbuf2, gbuf3,
             tbuf0, tbuf1, tbuf2, tbuf3,
             g0, g1, g2, g3, w0, w1, w2, w3):
        wid = lax.axis_index("s") * 2 + lax.axis_index("c")
        b0 = wid * _BPW
        pltpu.sync_copy(sidx_hbm.at[:, pl.ds(b0, _BPW)], idx_v)

        def shift_row(j, carry):
            for c in range(_BPW // 16):
                v = idx_v[j, pl.ds(c * 16, 16)]
                kidx_v[j, pl.ds(c * 16, 16)] = (
                    ((v >> _PACK_B) << (_PACK_B - 1))
                    | (v & (_PACK_N // 2 - 1)))
            return carry
        lax.fori_loop(0, _NUNIT, shift_row, 0)

        gbuf = (gbuf0, gbuf1, gbuf2, gbuf3)
        tbuf = (tbuf0, tbuf1, tbuf2, tbuf3)
        gsem = (g0, g1, g2, g3)
        wsem = (w0, w1, w2, w3)

        def start_gather(j, b):
            return pltpu.async_copy(wt_hbm.at[kidx_v.at[j]], gbuf[b], gsem[b])

        def start_writeback(j, b):
            return pltpu.async_copy(
                tbuf[b], out_hbm.at[j, :, pl.ds(b0, _BPW)], wsem[b])

        def wait_gather(j, b):
            pltpu.make_async_copy(wt_hbm.at[kidx_v.at[j]], gbuf[b],
                                  gsem[b]).wait()

        def wait_writeback(j, b):
            pltpu.make_async_copy(
                tbuf[b], out_hbm.at[j, :, pl.ds(b0, _BPW)], wsem[b]).wait()

        def transpose_unit(j, b):
            iota = lax.iota(jnp.int32, 16)
            for c in range(_BPW // 16):
                rows = iota + (c * 16)
                h6 = ((idx_v[j, pl.ds(c * 16, 16)] >> (_PACK_B - 1)) & 1) << 6

                @plsc.parallel_loop(0, _EMBED_DIM, step=1, unroll=8)
                def dloop(d):
                    v = plsc.load_gather(gbuf[b], [rows, h6 + d])
                    tbuf[b][d, pl.ds(c * 16, 16)] = v

        for b in range(4):
            start_gather(b, b)

        def unit_quad(s, carry):
            for b in (0, 1, 2, 3):
                j = 4 * s + b

                @pl.when((j >= 4) & (j < _NUNIT + 4))
                def _():
                    wait_writeback(j - 4, b)

                @pl.when(j < _NUNIT)
                def _():
                    wait_gather(j, b)
                    transpose_unit(j, b)

                @pl.when(j + 4 < _NUNIT)
                def _():
                    start_gather(j + 4, b)

                @pl.when(j < _NUNIT)
                def _():
                    start_writeback(j, b)
            return carry

        lax.fori_loop(0, _NUNIT // 4 + 2, unit_quad, 0)

    return body(wt_packed, sent_t)


def _tc_body(mask_t_ref, mtab_t_ref, m3_ref, pw_ref):
    m = mask_t_ref[...]
    q = lax.broadcasted_iota(jnp.int32, m.shape, 0)
    left = jnp.min(jnp.where(m == 1, q, jnp.int32(1 << 30)), axis=0,
                   keepdims=True)
    right = left + jnp.sum(m, axis=0, keepdims=True)
    d = jnp.where(q < left, left - q, jnp.where(q > right, q - right, 0))
    num = (100 - d) ** _POWER
    pw_ref[...] = num.astype(jnp.float32) / jnp.float32(100 ** _POWER)

    mt = mtab_t_ref[...]
    mt0 = mt[:, 0].reshape(1, _MASK_DIM, 1)
    mt1 = mt[:, 1].reshape(1, _MASK_DIM, 1)
    sel = (m == 1)[:, None, :]
    m3_ref[...] = jnp.where(sel, mt1, mt0)


def _tc_mask_pw(mask_t, mtab_t):
    grid = 8
    bb = _BATCH // grid
    return pl.pallas_call(
        _tc_body,
        grid=(grid,),
        in_specs=[
            pl.BlockSpec((_MAX_LEN, bb), lambda i: (0, i)),
            pl.BlockSpec((_MASK_DIM, 2), lambda i: (0, 0)),
        ],
        out_specs=[
            pl.BlockSpec((_MAX_LEN, _MASK_DIM, bb), lambda i: (0, 0, i)),
            pl.BlockSpec((_MAX_LEN, bb), lambda i: (0, i)),
        ],
        out_shape=[
            jax.ShapeDtypeStruct((_MAX_LEN, _MASK_DIM, _BATCH), jnp.float32),
            jax.ShapeDtypeStruct((_MAX_LEN, _BATCH), jnp.float32),
        ],
    )(mask_t, mtab_t)


def kernel(sent, mask, word_table, mask_table):
    wt_packed = _tc_pack(word_table)
    sent_t = sent.T
    mask_t = mask.T
    mtab_t = mask_table.T
    w3 = _sc_gather(wt_packed, sent_t)
    m3, pw_t = _tc_mask_pw(mask_t, mtab_t)
    sent_vec = jnp.transpose(w3, (2, 0, 1))
    mask_vec = jnp.transpose(m3, (2, 0, 1))
    position_weight = pw_t.T
    return (sent_vec, mask_vec, position_weight)
